# baseline (speedup 1.0000x reference)
import functools

import jax
import jax.numpy as jnp
from jax import lax
from jax.experimental import pallas as pl
from jax.experimental.pallas import tpu as pltpu
from jax.experimental.pallas import tpu_sc as plsc

NC = 2
NS = 16
NW = NC * NS

B = 4 * 8192
D = 1024
V = 3
L = 16
BPW = B // NW
C = 32
NCHUNK = BPW // C

_mesh = plsc.VectorSubcoreMesh(core_axis_name="c", subcore_axis_name="s")


@functools.partial(
    pl.kernel,
    mesh=_mesh,
    compiler_params=pltpu.CompilerParams(needs_layout_passes=False),
    out_type=jax.ShapeDtypeStruct((B, D), jnp.float32),
    scratch_types=[
        pltpu.SMEM((BPW,), jnp.int32),
        pltpu.VMEM((BPW + L,), jnp.int32),
        pltpu.VMEM((V, D), jnp.float32),
        pltpu.VMEM((C, D), jnp.float32),
        pltpu.VMEM((C, D), jnp.float32),
        pltpu.SemaphoreType.DMA,
        pltpu.SemaphoreType.DMA,
        pltpu.SemaphoreType.DMA,
    ],
)
def _embed_sc(table_hbm, idx_hbm, out_hbm, idx_v, idx_vv, table_v, rows0,
              rows1, sem0, sem1, gsem):
    wid = lax.axis_index("s") * NC + lax.axis_index("c")
    base = wid * BPW
    icopy = pltpu.async_copy(idx_hbm.at[pl.ds(base, BPW)],
                             idx_vv.at[pl.ds(0, BPW)], gsem)
    pltpu.sync_copy(table_hbm, table_v)
    icopy.wait()

    @plsc.parallel_loop(0, BPW, step=L, unroll=2)
    def scalarize(i):
        v = idx_vv[pl.ds(i, L)]
        for k in range(L):
            idx_v[i + k] = v[k]
    bufs = (rows0, rows1)
    sems = (sem0, sem1)
    ci = lax.iota(jnp.int32, L)

    def build_chunk(chunk_row, buf):
        Q = D // 4
        for h in range(4):
            t = [[table_v[r, pl.ds(h * Q + L * j, L)] for j in range(Q // L)]
                 for r in range(V)]

            @plsc.parallel_loop(0, C, unroll=4)
            def row(i):
                s = idx_v[chunk_row + i]
                e1 = s == 1
                e2 = s == 2
                for j in range(Q // L):
                    v = jnp.where(e2, t[2][j],
                                  jnp.where(e1, t[1][j], t[0][j]))
                    buf[i, pl.ds(h * Q + L * j, L)] = v

    def pair(c2, _):
        for b in range(2):
            c = 2 * c2 + b
            @pl.when(c2 >= 1)
            def _():
                pltpu.make_async_copy(bufs[b], out_hbm.at[pl.ds(base, C)],
                                      sems[b]).wait()
            build_chunk(c * C, bufs[b])
            pltpu.async_copy(bufs[b], out_hbm.at[pl.ds(base + c * C, C)],
                             sems[b])
        return 0

    lax.fori_loop(0, NCHUNK // 2, pair, 0)
    for b in range(2):
        pltpu.make_async_copy(bufs[b], out_hbm.at[pl.ds(base, C)],
                              sems[b]).wait()


def kernel(x, table):
    flat = x.reshape(-1).astype(jnp.int32)
    out = _embed_sc(table.astype(jnp.float32), flat)
    return out.reshape(x.shape[0], x.shape[1], D)


# speedup vs baseline: 1.7871x; 1.7871x over previous
import functools

import jax
import jax.numpy as jnp
from jax import lax
from jax.experimental import pallas as pl
from jax.experimental.pallas import tpu as pltpu
from jax.experimental.pallas import tpu_sc as plsc

NC = 2
NS = 16
NW = NC * NS

B = 4 * 8192
D = 1024
V = 3
L = 16
BPW = B // NW
C = 32
NCHUNK = BPW // C

_mesh = plsc.VectorSubcoreMesh(core_axis_name="c", subcore_axis_name="s")


@functools.partial(
    pl.kernel,
    mesh=_mesh,
    compiler_params=pltpu.CompilerParams(needs_layout_passes=False),
    out_type=jax.ShapeDtypeStruct((B, D), jnp.float32),
    scratch_types=[
        pltpu.SMEM((BPW,), jnp.int32),
        pltpu.VMEM((BPW + L,), jnp.int32),
        pltpu.VMEM((V, D), jnp.float32),
        pltpu.VMEM((C, D), jnp.float32),
        pltpu.VMEM((C, D), jnp.float32),
        pltpu.SemaphoreType.DMA,
        pltpu.SemaphoreType.DMA,
        pltpu.SemaphoreType.DMA,
    ],
)
def _embed_sc(table_hbm, idx_hbm, out_hbm, idx_v, idx_vv, table_v, rows0,
              rows1, sem0, sem1, gsem):
    wid = lax.axis_index("s") * NC + lax.axis_index("c")
    base = wid * BPW
    icopy = pltpu.async_copy(idx_hbm.at[pl.ds(base, BPW)],
                             idx_vv.at[pl.ds(0, BPW)], gsem)
    pltpu.sync_copy(table_hbm, table_v)
    icopy.wait()

    @plsc.parallel_loop(0, BPW, step=L, unroll=2)
    def scalarize(i):
        v = idx_vv[pl.ds(i, L)]
        for k in range(L):
            idx_v[i + k] = v[k]
    bufs = (rows0, rows1)
    sems = (sem0, sem1)
    ci = lax.iota(jnp.int32, L)

    def build_chunk(chunk_row, buf):
        Q = D // 4
        zero = jnp.zeros((L,), jnp.float32)
        for h in range(4):
            t = [[table_v[r, pl.ds(h * Q + L * j, L)] for j in range(Q // L)]
                 for r in (1, 2)]

            @plsc.parallel_loop(0, C, unroll=2)
            def row(i):
                s = idx_v[chunk_row + i]
                e1 = s == 1
                e2 = s == 2
                for j in range(Q // L):
                    v = jnp.where(e2, t[1][j],
                                  jnp.where(e1, t[0][j], zero))
                    buf[i, pl.ds(h * Q + L * j, L)] = v

    def pair(c2, _):
        for b in range(2):
            c = 2 * c2 + b
            @pl.when(c2 >= 1)
            def _():
                pltpu.make_async_copy(bufs[b], out_hbm.at[pl.ds(base, C)],
                                      sems[b]).wait()
            build_chunk(c * C, bufs[b])
            pltpu.async_copy(bufs[b], out_hbm.at[pl.ds(base + c * C, C)],
                             sems[b])
        return 0

    lax.fori_loop(0, NCHUNK // 2, pair, 0)
    for b in range(2):
        pltpu.make_async_copy(bufs[b], out_hbm.at[pl.ds(base, C)],
                              sems[b]).wait()


def kernel(x, table):
    flat = x.reshape(-1).astype(jnp.int32)
    out = _embed_sc(table.astype(jnp.float32), flat)
    return out.reshape(x.shape[0], x.shape[1], D)
